# baseline (speedup 1.0000x reference)
import jax
import jax.numpy as jnp
from jax import lax
from jax.experimental import pallas as pl
from jax.experimental.pallas import tpu as pltpu
from jax.experimental.pallas import tpu_sc as plsc

D = 64
DP = 128
CUST = 4096
R_TAB = 100000
BLK = 5000
NC, NS = 2, 16
NW = NC * NS
IDX_W = 128
ROWS_T = CUST // NS
N_CI = 2048
CI_T = N_CI // NS


def _matmul_body(ct_ref, w_ref, b_ref, t_ref):
    t_ref[...] = lax.dot_general(
        ct_ref[...], w_ref[...], (((1,), (1,)), ((), ())),
        preferred_element_type=jnp.float32,
        precision=lax.Precision.HIGHEST,
    ) + b_ref[...]


def _patch_body(reg_hbm, t_hbm, ci_hbm, b_hbm, p_hbm,
                rows_v, trows_v, ci_v, b_v, sem):
    c = lax.axis_index("c")
    s = lax.axis_index("s")

    @pl.when(c == 0)
    def _():
        r0 = s * ROWS_T
        pltpu.sync_copy(reg_hbm.at[pl.ds(r0, ROWS_T)], rows_v)
        pltpu.sync_copy(b_hbm, b_v)
        bvals = [b_v[pl.ds(16 * k, 16)] for k in range(4)]

        def addb(r, carry):
            for k in range(4):
                sl = pl.ds(16 * k, 16)
                rows_v[r, sl] = rows_v[r, sl] + bvals[k]
            return carry

        lax.fori_loop(0, ROWS_T, addb, 0)
        pltpu.sync_copy(rows_v, p_hbm.at[pl.ds(r0, ROWS_T)])
        plsc.subcore_barrier()
        pltpu.sync_copy(ci_hbm.at[pl.ds(s * CI_T, CI_T)], ci_v)
        pltpu.async_copy(t_hbm.at[ci_v], trows_v, sem).wait()
        pltpu.async_copy(trows_v, p_hbm.at[ci_v], sem).wait()


def _merge_body(reg_ref, p_ref, b_ref, out_ref):
    i = pl.program_id(0)
    out_ref[:, 0:D] = reg_ref[...] + b_ref[...]
    out_ref[:, D:DP] = jnp.zeros((BLK, DP - D), jnp.float32)

    @pl.when(i == 0)
    def _():
        out_ref[0:CUST, 0:D] = p_ref[...]


def _gather_body(tab_hbm, xt_hbm, out_hbm, idx_v, rows_v, trans_v, sem):
    c = lax.axis_index("c")
    s = lax.axis_index("s")
    nblk_b = xt_hbm.shape[1] // IDX_W
    blocks = xt_hbm.shape[0] * nblk_b
    per_w = blocks // NW
    wid = s * NC + c
    jota = lax.iota(jnp.int32, 16)

    def step(g, carry):
        bid = wid * per_w + g
        l = bid // nblk_b
        b0 = (bid % nblk_b) * IDX_W
        pltpu.sync_copy(xt_hbm.at[l, pl.ds(b0, IDX_W)], idx_v)
        pltpu.async_copy(tab_hbm.at[idx_v], rows_v, sem).wait()

        def trans(d, carry2):
            cidx = jnp.full((16,), d, jnp.int32)
            for j0 in range(0, IDX_W, 16):
                vals = plsc.load_gather(rows_v, [jota + j0, cidx])
                trans_v[d, pl.ds(j0, 16)] = vals
            return carry2

        lax.fori_loop(0, D, trans, 0)
        pltpu.sync_copy(trans_v, out_hbm.at[l, :, pl.ds(b0, IDX_W)])
        return carry

    lax.fori_loop(0, per_w, step, 0)


def kernel(x, custom_indices, custom_table, regular_table, W, b):
    B, L = x.shape
    xt = jnp.transpose(x).astype(jnp.int32)
    b1 = b.astype(jnp.float32)
    b2 = b1.reshape(1, D)
    ci = custom_indices.reshape(N_CI).astype(jnp.int32)

    t_tab = pl.pallas_call(
        _matmul_body,
        out_shape=jax.ShapeDtypeStruct((CUST, D), jnp.float32),
    )(custom_table, W, b2)

    sc_mesh = plsc.VectorSubcoreMesh(core_axis_name="c", subcore_axis_name="s")

    patch = pl.kernel(
        _patch_body,
        out_type=jax.ShapeDtypeStruct((CUST, D), jnp.float32),
        mesh=sc_mesh,
        compiler_params=pltpu.CompilerParams(use_tc_tiling_on_sc=False),
        scratch_types=[
            pltpu.VMEM((ROWS_T, D), jnp.float32),
            pltpu.VMEM((CI_T, D), jnp.float32),
            pltpu.VMEM((CI_T,), jnp.int32),
            pltpu.VMEM((D,), jnp.float32),
            pltpu.SemaphoreType.DMA,
        ],
    )
    reg4k = lax.slice(regular_table, (0, 0), (CUST, D))
    p_tab = patch(reg4k, t_tab, ci, b1)

    merged = pl.pallas_call(
        _merge_body,
        grid=(R_TAB // BLK,),
        in_specs=[
            pl.BlockSpec((BLK, D), lambda i: (i, 0)),
            pl.BlockSpec((CUST, D), lambda i: (0, 0)),
            pl.BlockSpec((1, D), lambda i: (0, 0)),
        ],
        out_specs=pl.BlockSpec((BLK, DP), lambda i: (i, 0)),
        out_shape=jax.ShapeDtypeStruct((R_TAB, DP), jnp.float32),
    )(regular_table, p_tab, b2)

    gather = pl.kernel(
        _gather_body,
        out_type=jax.ShapeDtypeStruct((L, D, B), jnp.float32),
        mesh=sc_mesh,
        compiler_params=pltpu.CompilerParams(
            use_tc_tiling_on_sc=True, needs_layout_passes=False),
        scratch_types=[
            pltpu.VMEM((IDX_W,), jnp.int32),
            pltpu.VMEM((IDX_W, DP), jnp.float32),
            pltpu.VMEM((D, IDX_W), jnp.float32),
            pltpu.SemaphoreType.DMA,
        ],
    )
    out3 = gather(merged, xt)
    return jnp.transpose(out3, (2, 0, 1))


# speedup vs baseline: 1.3192x; 1.3192x over previous
import jax
import jax.numpy as jnp
from jax import lax
from jax.experimental import pallas as pl
from jax.experimental.pallas import tpu as pltpu
from jax.experimental.pallas import tpu_sc as plsc

D = 64
DP = 128
CUST = 4096
R_TAB = 100000
BLK = 5000
NC, NS = 2, 16
NW = NC * NS
IDX_W = 128
ROWS_T = CUST // NS
N_CI = 2048
CI_T = N_CI // NS


def _matmul_body(ct_ref, w_ref, b_ref, t_ref):
    t_ref[...] = lax.dot_general(
        ct_ref[...], w_ref[...], (((1,), (1,)), ((), ())),
        preferred_element_type=jnp.float32,
        precision=lax.Precision.HIGHEST,
    ) + b_ref[...]


def _patch_body(reg_hbm, t_hbm, ci_hbm, b_hbm, p_hbm,
                rows_v, trows_v, ci_v, b_v, sem):
    c = lax.axis_index("c")
    s = lax.axis_index("s")

    @pl.when(c == 0)
    def _():
        r0 = s * ROWS_T
        pltpu.sync_copy(reg_hbm.at[pl.ds(r0, ROWS_T)], rows_v)
        pltpu.sync_copy(b_hbm, b_v)
        bvals = [b_v[pl.ds(16 * k, 16)] for k in range(4)]

        def addb(r, carry):
            for k in range(4):
                sl = pl.ds(16 * k, 16)
                rows_v[r, sl] = rows_v[r, sl] + bvals[k]
            return carry

        lax.fori_loop(0, ROWS_T, addb, 0)
        pltpu.sync_copy(rows_v, p_hbm.at[pl.ds(r0, ROWS_T)])
        plsc.subcore_barrier()
        pltpu.sync_copy(ci_hbm.at[pl.ds(s * CI_T, CI_T)], ci_v)
        pltpu.async_copy(t_hbm.at[ci_v], trows_v, sem).wait()
        pltpu.async_copy(trows_v, p_hbm.at[ci_v], sem).wait()


def _merge_body(reg_ref, p_ref, b_ref, out_ref):
    i = pl.program_id(0)
    out_ref[:, 0:D] = reg_ref[...] + b_ref[...]
    out_ref[:, D:DP] = jnp.zeros((BLK, DP - D), jnp.float32)

    @pl.when(i == 0)
    def _():
        out_ref[0:CUST, 0:D] = p_ref[...]


def _gather_body(tab_hbm, xt_hbm, out_hbm, idx_v,
                 rows0, rows1, trans0, trans1, gsem, wsem):
    c = lax.axis_index("c")
    s = lax.axis_index("s")
    nblk_b = xt_hbm.shape[1] // IDX_W
    per_w = nblk_b // NW
    n_l = xt_hbm.shape[0]
    wid = s * NC + c
    jota = lax.iota(jnp.int32, 16)

    def transpose(rows_v, trans_v):
        def trans(d, carry2):
            cidx = jnp.full((16,), d, jnp.int32)
            for j0 in range(0, IDX_W, 16):
                vals = plsc.load_gather(rows_v, [jota + j0, cidx])
                trans_v[d, pl.ds(j0, 16)] = vals
            return carry2

        lax.fori_loop(0, D, trans, 0)

    def drain_g():
        pltpu.make_async_copy(tab_hbm.at[pl.ds(0, IDX_W)], rows0, gsem).wait()

    def drain_w(b0):
        pltpu.make_async_copy(
            trans0, out_hbm.at[0, :, pl.ds(b0, IDX_W)], wsem).wait()

    def do_t(t, carry):
        b0 = (wid * per_w + t) * IDX_W
        pltpu.sync_copy(xt_hbm.at[:, pl.ds(b0, IDX_W)], idx_v)
        pltpu.async_copy(tab_hbm.at[idx_v.at[0]], rows0, gsem)

        def pair(i, carry2):
            l0 = 2 * i
            pltpu.async_copy(tab_hbm.at[idx_v.at[l0 + 1]], rows1, gsem)
            drain_g()

            @pl.when(i > 0)
            def _():
                drain_w(b0)
            transpose(rows0, trans0)
            pltpu.async_copy(trans0, out_hbm.at[l0, :, pl.ds(b0, IDX_W)], wsem)

            @pl.when(l0 + 2 < n_l)
            def _():
                pltpu.async_copy(tab_hbm.at[idx_v.at[l0 + 2]], rows0, gsem)
            drain_g()

            @pl.when(i > 0)
            def _():
                drain_w(b0)
            transpose(rows1, trans1)
            pltpu.async_copy(
                trans1, out_hbm.at[l0 + 1, :, pl.ds(b0, IDX_W)], wsem)
            return carry2

        lax.fori_loop(0, n_l // 2, pair, 0)
        drain_w(b0)
        drain_w(b0)
        return carry

    lax.fori_loop(0, per_w, do_t, 0)


def kernel(x, custom_indices, custom_table, regular_table, W, b):
    B, L = x.shape
    xt = jnp.transpose(x).astype(jnp.int32)
    b1 = b.astype(jnp.float32)
    b2 = b1.reshape(1, D)
    ci = custom_indices.reshape(N_CI).astype(jnp.int32)

    t_tab = pl.pallas_call(
        _matmul_body,
        out_shape=jax.ShapeDtypeStruct((CUST, D), jnp.float32),
    )(custom_table, W, b2)

    sc_mesh = plsc.VectorSubcoreMesh(core_axis_name="c", subcore_axis_name="s")

    patch = pl.kernel(
        _patch_body,
        out_type=jax.ShapeDtypeStruct((CUST, D), jnp.float32),
        mesh=sc_mesh,
        compiler_params=pltpu.CompilerParams(use_tc_tiling_on_sc=False),
        scratch_types=[
            pltpu.VMEM((ROWS_T, D), jnp.float32),
            pltpu.VMEM((CI_T, D), jnp.float32),
            pltpu.VMEM((CI_T,), jnp.int32),
            pltpu.VMEM((D,), jnp.float32),
            pltpu.SemaphoreType.DMA,
        ],
    )
    reg4k = lax.slice(regular_table, (0, 0), (CUST, D))
    p_tab = patch(reg4k, t_tab, ci, b1)

    merged = pl.pallas_call(
        _merge_body,
        grid=(R_TAB // BLK,),
        in_specs=[
            pl.BlockSpec((BLK, D), lambda i: (i, 0)),
            pl.BlockSpec((CUST, D), lambda i: (0, 0)),
            pl.BlockSpec((1, D), lambda i: (0, 0)),
        ],
        out_specs=pl.BlockSpec((BLK, DP), lambda i: (i, 0)),
        out_shape=jax.ShapeDtypeStruct((R_TAB, DP), jnp.float32),
    )(regular_table, p_tab, b2)

    gather = pl.kernel(
        _gather_body,
        out_type=jax.ShapeDtypeStruct((L, D, B), jnp.float32),
        mesh=sc_mesh,
        compiler_params=pltpu.CompilerParams(
            use_tc_tiling_on_sc=True, needs_layout_passes=False),
        scratch_types=[
            pltpu.VMEM((L, IDX_W), jnp.int32),
            pltpu.VMEM((IDX_W, DP), jnp.float32),
            pltpu.VMEM((IDX_W, DP), jnp.float32),
            pltpu.VMEM((D, IDX_W), jnp.float32),
            pltpu.VMEM((D, IDX_W), jnp.float32),
            pltpu.SemaphoreType.DMA,
            pltpu.SemaphoreType.DMA,
        ],
    )
    out3 = gather(merged, xt)
    return jnp.transpose(out3, (2, 0, 1))
